# baseline (speedup 1.0000x reference)
import functools

import jax
import jax.numpy as jnp
from jax import lax
from jax.experimental import pallas as pl
from jax.experimental.pallas import tpu as pltpu
from jax.experimental.pallas import tpu_sc as plsc

N = 10000
D = 128
E = 320000

NC = 2
NS = 16
NW = NC * NS
EPW = E // NW
GK = 40
SK = 80
NGC = EPW // GK
NSC = EPW // SK
NSLOT = 6
SPG = SK // GK
RPT = N // NS
EPRE = EPW + 112
NPAD = 10240

_mesh = plsc.VectorSubcoreMesh(
    core_axis_name="c", subcore_axis_name="s", num_cores=NC, num_subcores=NS
)


@functools.partial(
    pl.kernel,
    out_type=jax.ShapeDtypeStruct((NC * NPAD,), jnp.float32),
    mesh=_mesh,
    scratch_types=[
        pltpu.VMEM((EPRE,), jnp.int32),
        pltpu.VMEM((8, SK), jnp.int32),
        pltpu.VMEM((SK,), jnp.float32),
        pltpu.VMEM((1024,), jnp.float32),
        pltpu.VMEM_SHARED((NPAD,), jnp.float32),
        pltpu.SemaphoreType.DMA((2,)),
    ],
)
def _sc_degree(ei_hbm, out_hbm, dsts_v, idxp_v, ones_v, zeros_v, acc_sh, semd):
    c = lax.axis_index("c")
    s = lax.axis_index("s")
    wid = s * NC + c

    for i in range(SK // 16):
        ones_v[pl.ds(i * 16, 16)] = jnp.full((16,), 1.0, jnp.float32)
    for i in range(1024 // 16):
        zeros_v[pl.ds(i * 16, 16)] = jnp.zeros((16,), jnp.float32)

    @pl.when(s < 10)
    def _():
        off = pl.multiple_of(s * 1024, 8)
        pltpu.sync_copy(zeros_v, acc_sh.at[pl.ds(off, 1024)])

    base = wid * EPW
    local = lax.rem(base, 128)
    a128 = pl.multiple_of(base - local, 128)
    pltpu.sync_copy(ei_hbm.at[1, pl.ds(a128, EPRE)], dsts_v)
    plsc.subcore_barrier()

    def transform(t, b):
        for k in range(SK // 16):
            off = pl.multiple_of(local + t * SK + k * 16, 8)
            n = dsts_v[pl.ds(off, 16)]
            q = (n.astype(jnp.float32) * 0.001).astype(jnp.int32)
            idxp_v[b, pl.ds(k * 16, 16)] = n + 24 * q

    def sstart(b):
        pltpu.async_copy(ones_v, acc_sh.at[idxp_v.at[b]], semd.at[b], add=True)

    def swaitd(b):
        pltpu.make_async_copy(ones_v, acc_sh.at[idxp_v.at[b]], semd.at[b]).wait()

    for b in range(2):
        transform(b, b)
        sstart(b)

    def chunk2(g, carry):
        for b in range(2):
            swaitd(b)
            transform(2 * g + 2 + b, b)
            sstart(b)
        return carry

    lax.fori_loop(0, (NSC - 3) // 2, chunk2, 0)
    swaitd(0)
    transform(NSC - 1, 0)
    sstart(0)
    swaitd(0)
    swaitd(1)
    plsc.subcore_barrier()

    @pl.when(s < 10)
    def _():
        off = pl.multiple_of(s * 1024, 8)
        dst_off = pl.multiple_of(c * NPAD + s * 1024, 8)
        pltpu.sync_copy(acc_sh.at[pl.ds(off, 1024)], zeros_v)
        pltpu.sync_copy(zeros_v, out_hbm.at[pl.ds(dst_off, 1024)])


@functools.partial(
    pl.kernel,
    out_type=jax.ShapeDtypeStruct((NC, N, D), jnp.float32),
    mesh=_mesh,
    scratch_types=[
        pltpu.VMEM((EPRE,), jnp.int32),
        pltpu.VMEM((EPRE,), jnp.int32),
        pltpu.VMEM((NSLOT * GK, D), jnp.float32),
        pltpu.VMEM_SHARED((N, D), jnp.float32),
        pltpu.SemaphoreType.DMA((NSLOT,)),
    ],
)
def _sc_edge_scatter(g_hbm, ei_hbm, out_hbm, srcs_v, dsts_v, rows_v, acc_sh, sems):
    c = lax.axis_index("c")
    s = lax.axis_index("s")
    wid = s * NC + c

    nring = NSLOT * GK
    start = pl.multiple_of(s * RPT - s % 8, 8)

    @pl.when(c == 0)
    def _():
        for i in range(3):
            off = pl.multiple_of(jnp.minimum(start + i * nring, N - nring), 8)
            pltpu.sync_copy(g_hbm.at[pl.ds(off, nring)], rows_v)
            pltpu.sync_copy(rows_v, acc_sh.at[pl.ds(off, nring)])

    @pl.when(c == 1)
    def _():
        def zrow(i, carry):
            for g in range(D // 16):
                rows_v[i, pl.ds(g * 16, 16)] = jnp.zeros((16,), jnp.float32)
            return carry

        lax.fori_loop(0, nring, zrow, 0)
        for i in range(3):
            off = pl.multiple_of(jnp.minimum(start + i * nring, N - nring), 8)
            pltpu.sync_copy(rows_v, acc_sh.at[pl.ds(off, nring)])

    base = wid * EPW
    local = lax.rem(base, 128)
    a128 = pl.multiple_of(base - local, 128)
    pltpu.sync_copy(ei_hbm.at[0, pl.ds(a128, EPRE)], srcs_v)
    pltpu.sync_copy(ei_hbm.at[1, pl.ds(a128, EPRE)], dsts_v)
    plsc.subcore_barrier()

    def gath(j, slot):
        idx = srcs_v.at[pl.ds(pl.multiple_of(local + j * GK, 8), GK)]
        pltpu.async_copy(
            g_hbm.at[idx], rows_v.at[pl.ds(slot * GK, GK)], sems.at[slot]
        )

    def gwait(slot):
        pltpu.make_async_copy(
            g_hbm.at[pl.ds(0, GK)], rows_v.at[pl.ds(slot * GK, GK)], sems.at[slot]
        ).wait()

    def scat(p, slot0):
        idx = dsts_v.at[pl.ds(pl.multiple_of(local + p * SK, 8), SK)]
        pltpu.sync_copy(rows_v.at[pl.ds(slot0 * GK, SK)], acc_sh.at[idx], add=True)

    for j in range(NSLOT - 1):
        gath(j, j)

    def group(g, carry):
        for sub in range(NSLOT // SPG):
            for q in range(SPG):
                t = NSLOT * g + SPG * sub + q
                slot = SPG * sub + q
                gwait(slot)

                @pl.when(t + NSLOT - 1 < NGC)
                def _():
                    gath(t + NSLOT - 1, (slot + NSLOT - 1) % NSLOT)

            scat((NSLOT // SPG) * g + sub, SPG * sub)
        return carry

    NFULL = NGC // NSLOT
    lax.fori_loop(0, NFULL, group, 0)
    for q in range(2 * SPG):
        gwait(q)
    scat(NSC - 2, 0)
    scat(NSC - 1, SPG)

    plsc.subcore_barrier()

    for i in range(6):
        sz = 120 if i < 5 else 632 - 5 * 120
        off = pl.multiple_of(start + i * 120, 8)
        stg = rows_v.at[pl.ds((i % 2) * 120, sz)]
        if i >= 2:
            pltpu.make_async_copy(
                acc_sh.at[pl.ds(off, 120)],
                rows_v.at[pl.ds((i % 2) * 120, 120)],
                sems.at[i % 2],
            ).wait()
        pltpu.sync_copy(acc_sh.at[pl.ds(off, sz)], stg)
        pltpu.async_copy(stg, out_hbm.at[c, pl.ds(off, sz)], sems.at[i % 2])
    pltpu.make_async_copy(
        acc_sh.at[pl.ds(start, 120)], rows_v.at[pl.ds(0, 120)], sems.at[0]
    ).wait()
    pltpu.make_async_copy(
        acc_sh.at[pl.ds(start, 32)], rows_v.at[pl.ds(120, 32)], sems.at[1]
    ).wait()


BR = 1000


def _deg_block(degp_ref, i):
    d0 = degp_ref[pl.ds(pl.multiple_of(i * 1024, 128), 1024)][:BR]
    d1 = degp_ref[pl.ds(pl.multiple_of(NPAD + i * 1024, 128), 1024)][:BR]
    return lax.rsqrt(1.0 + d0 + d1).reshape(BR, 1)


def _tc1_body(x_ref, w_ref, degp_ref, g_ref):
    dis = _deg_block(degp_ref, pl.program_id(0))
    h = jnp.dot(x_ref[...], w_ref[...], preferred_element_type=jnp.float32)
    g_ref[...] = h * dis


def _tc1(x, W1, degp):
    return pl.pallas_call(
        _tc1_body,
        grid=(N // BR,),
        in_specs=[
            pl.BlockSpec((BR, D), lambda i: (i, 0)),
            pl.BlockSpec((D, D), lambda i: (0, 0)),
            pl.BlockSpec((NC * NPAD,), lambda i: (0,)),
        ],
        out_specs=pl.BlockSpec((BR, D), lambda i: (i, 0)),
        out_shape=jax.ShapeDtypeStruct((N, D), jnp.float32),
    )(x, W1, degp)


def _tc2_body(sp_ref, degp_ref, b_ref, w_ref, g2_ref):
    dis = _deg_block(degp_ref, pl.program_id(0))
    t = dis * (sp_ref[0] + sp_ref[1]) + b_ref[...]
    t = jnp.maximum(t, 0.0)
    h2 = jnp.dot(t, w_ref[...], preferred_element_type=jnp.float32)
    g2_ref[...] = h2 * dis


def _tc2(sp, degp, b1, W2):
    return pl.pallas_call(
        _tc2_body,
        grid=(N // BR,),
        in_specs=[
            pl.BlockSpec((NC, BR, D), lambda i: (0, i, 0)),
            pl.BlockSpec((NC * NPAD,), lambda i: (0,)),
            pl.BlockSpec((1, D), lambda i: (0, 0)),
            pl.BlockSpec((D, D), lambda i: (0, 0)),
        ],
        out_specs=pl.BlockSpec((BR, D), lambda i: (i, 0)),
        out_shape=jax.ShapeDtypeStruct((N, D), jnp.float32),
    )(sp, degp, b1, W2)


def _tc3_body(sp_ref, degp_ref, b_ref, out_ref):
    dis = _deg_block(degp_ref, pl.program_id(0))
    out_ref[...] = dis * (sp_ref[0] + sp_ref[1]) + b_ref[...]


def _tc3(sp, degp, b2):
    return pl.pallas_call(
        _tc3_body,
        grid=(N // BR,),
        in_specs=[
            pl.BlockSpec((NC, BR, D), lambda i: (0, i, 0)),
            pl.BlockSpec((NC * NPAD,), lambda i: (0,)),
            pl.BlockSpec((1, D), lambda i: (0, 0)),
        ],
        out_specs=pl.BlockSpec((BR, D), lambda i: (i, 0)),
        out_shape=jax.ShapeDtypeStruct((N, D), jnp.float32),
    )(sp, degp, b2)


def kernel(x, edge_index, W1, b1, W2, b2):
    degp = _sc_degree(edge_index)
    g1 = _tc1(x, W1, degp)
    s1 = _sc_edge_scatter(g1, edge_index)
    g2 = _tc2(s1, degp, b1.reshape(1, D), W2)
    s2 = _sc_edge_scatter(g2, edge_index)
    out = _tc3(s2, degp, b2.reshape(1, D))
    return out


# speedup vs baseline: 1.0299x; 1.0299x over previous
import functools

import jax
import jax.numpy as jnp
from jax import lax
from jax.experimental import pallas as pl
from jax.experimental.pallas import tpu as pltpu
from jax.experimental.pallas import tpu_sc as plsc

N = 10000
D = 128
E = 320000

NC = 2
NS = 16
NW = NC * NS
EPW = E // NW
GK = 40
SK = 80
NGC = EPW // GK
NSC = EPW // SK
NSLOT = 6
SPG = SK // GK
RPT = N // NS
EPRE = EPW + 112
NPAD = 10240

_mesh = plsc.VectorSubcoreMesh(
    core_axis_name="c", subcore_axis_name="s", num_cores=NC, num_subcores=NS
)


@functools.partial(
    pl.kernel,
    out_type=jax.ShapeDtypeStruct((NC * NPAD,), jnp.float32),
    mesh=_mesh,
    scratch_types=[
        pltpu.VMEM((EPRE,), jnp.int32),
        pltpu.VMEM((8, SK), jnp.int32),
        pltpu.VMEM((SK,), jnp.float32),
        pltpu.VMEM((1024,), jnp.float32),
        pltpu.VMEM_SHARED((NPAD,), jnp.float32),
        pltpu.SemaphoreType.DMA((2,)),
    ],
)
def _sc_degree(ei_hbm, out_hbm, dsts_v, idxp_v, ones_v, zeros_v, acc_sh, semd):
    c = lax.axis_index("c")
    s = lax.axis_index("s")
    wid = s * NC + c

    for i in range(SK // 16):
        ones_v[pl.ds(i * 16, 16)] = jnp.full((16,), 1.0, jnp.float32)
    for i in range(1024 // 16):
        zeros_v[pl.ds(i * 16, 16)] = jnp.zeros((16,), jnp.float32)

    @pl.when(s < 10)
    def _():
        off = pl.multiple_of(s * 1024, 8)
        pltpu.sync_copy(zeros_v, acc_sh.at[pl.ds(off, 1024)])

    base = wid * EPW
    local = lax.rem(base, 128)
    a128 = pl.multiple_of(base - local, 128)
    pltpu.sync_copy(ei_hbm.at[1, pl.ds(a128, EPRE)], dsts_v)
    plsc.subcore_barrier()

    def transform(t, b):
        for k in range(SK // 16):
            off = pl.multiple_of(local + t * SK + k * 16, 8)
            n = dsts_v[pl.ds(off, 16)]
            q = (n.astype(jnp.float32) * 0.001).astype(jnp.int32)
            idxp_v[b, pl.ds(k * 16, 16)] = n + 24 * q

    def sstart(b):
        pltpu.async_copy(ones_v, acc_sh.at[idxp_v.at[b]], semd.at[b], add=True)

    def swaitd(b):
        pltpu.make_async_copy(ones_v, acc_sh.at[idxp_v.at[b]], semd.at[b]).wait()

    for b in range(2):
        transform(b, b)
        sstart(b)

    def chunk2(g, carry):
        for b in range(2):
            swaitd(b)
            transform(2 * g + 2 + b, b)
            sstart(b)
        return carry

    lax.fori_loop(0, (NSC - 3) // 2, chunk2, 0)
    swaitd(0)
    transform(NSC - 1, 0)
    sstart(0)
    swaitd(0)
    swaitd(1)
    plsc.subcore_barrier()

    @pl.when(s < 10)
    def _():
        off = pl.multiple_of(s * 1024, 8)
        dst_off = pl.multiple_of(c * NPAD + s * 1024, 8)
        pltpu.sync_copy(acc_sh.at[pl.ds(off, 1024)], zeros_v)
        pltpu.sync_copy(zeros_v, out_hbm.at[pl.ds(dst_off, 1024)])


@functools.partial(
    pl.kernel,
    out_type=jax.ShapeDtypeStruct((NC, N, D), jnp.float32),
    mesh=_mesh,
    scratch_types=[
        pltpu.VMEM((EPRE,), jnp.int32),
        pltpu.VMEM((EPRE,), jnp.int32),
        pltpu.VMEM((NSLOT * GK, D), jnp.float32),
        pltpu.VMEM_SHARED((N, D), jnp.float32),
        pltpu.SemaphoreType.DMA((NSLOT,)),
    ],
)
def _sc_edge_scatter(g_hbm, ei_hbm, out_hbm, srcs_v, dsts_v, rows_v, acc_sh, sems):
    c = lax.axis_index("c")
    s = lax.axis_index("s")
    wid = s * NC + c

    def zrow(i, carry):
        for g in range(D // 16):
            rows_v[i, pl.ds(g * 16, 16)] = jnp.zeros((16,), jnp.float32)
        return carry

    nring = NSLOT * GK
    lax.fori_loop(0, nring, zrow, 0)
    start = pl.multiple_of(s * RPT - s % 8, 8)
    for i in range(3):
        off = pl.multiple_of(jnp.minimum(start + i * nring, N - nring), 8)
        pltpu.sync_copy(rows_v, acc_sh.at[pl.ds(off, nring)])

    base = wid * EPW
    local = lax.rem(base, 128)
    a128 = pl.multiple_of(base - local, 128)
    pltpu.sync_copy(ei_hbm.at[0, pl.ds(a128, EPRE)], srcs_v)
    pltpu.sync_copy(ei_hbm.at[1, pl.ds(a128, EPRE)], dsts_v)
    plsc.subcore_barrier()

    def gath(j, slot):
        idx = srcs_v.at[pl.ds(pl.multiple_of(local + j * GK, 8), GK)]
        pltpu.async_copy(
            g_hbm.at[idx], rows_v.at[pl.ds(slot * GK, GK)], sems.at[slot]
        )

    def gwait(slot):
        pltpu.make_async_copy(
            g_hbm.at[pl.ds(0, GK)], rows_v.at[pl.ds(slot * GK, GK)], sems.at[slot]
        ).wait()

    def scat(p, slot0):
        idx = dsts_v.at[pl.ds(pl.multiple_of(local + p * SK, 8), SK)]
        pltpu.sync_copy(rows_v.at[pl.ds(slot0 * GK, SK)], acc_sh.at[idx], add=True)

    for j in range(NSLOT - 1):
        gath(j, j)

    def group(g, carry):
        for sub in range(NSLOT // SPG):
            for q in range(SPG):
                t = NSLOT * g + SPG * sub + q
                slot = SPG * sub + q
                gwait(slot)

                @pl.when(t + NSLOT - 1 < NGC)
                def _():
                    gath(t + NSLOT - 1, (slot + NSLOT - 1) % NSLOT)

            scat((NSLOT // SPG) * g + sub, SPG * sub)
        return carry

    NFULL = NGC // NSLOT
    lax.fori_loop(0, NFULL, group, 0)
    for q in range(2 * SPG):
        gwait(q)
    scat(NSC - 2, 0)
    scat(NSC - 1, SPG)

    plsc.subcore_barrier()

    for i in range(6):
        sz = 120 if i < 5 else 632 - 5 * 120
        off = pl.multiple_of(start + i * 120, 8)
        stg = rows_v.at[pl.ds((i % 2) * 120, sz)]
        if i >= 2:
            pltpu.make_async_copy(
                acc_sh.at[pl.ds(off, 120)],
                rows_v.at[pl.ds((i % 2) * 120, 120)],
                sems.at[i % 2],
            ).wait()
        pltpu.sync_copy(acc_sh.at[pl.ds(off, sz)], stg)
        pltpu.async_copy(stg, out_hbm.at[c, pl.ds(off, sz)], sems.at[i % 2])
    pltpu.make_async_copy(
        acc_sh.at[pl.ds(start, 120)], rows_v.at[pl.ds(0, 120)], sems.at[0]
    ).wait()
    pltpu.make_async_copy(
        acc_sh.at[pl.ds(start, 32)], rows_v.at[pl.ds(120, 32)], sems.at[1]
    ).wait()


BR = 1000


def _deg_block(degp_ref, i):
    d0 = degp_ref[pl.ds(pl.multiple_of(i * 1024, 128), 1024)][:BR]
    d1 = degp_ref[pl.ds(pl.multiple_of(NPAD + i * 1024, 128), 1024)][:BR]
    return lax.rsqrt(1.0 + d0 + d1).reshape(BR, 1)


def _tc1_body(x_ref, w_ref, degp_ref, g_ref):
    dis = _deg_block(degp_ref, pl.program_id(0))
    h = jnp.dot(x_ref[...], w_ref[...], preferred_element_type=jnp.float32)
    g_ref[...] = h * dis


def _tc1(x, W1, degp):
    return pl.pallas_call(
        _tc1_body,
        grid=(N // BR,),
        in_specs=[
            pl.BlockSpec((BR, D), lambda i: (i, 0)),
            pl.BlockSpec((D, D), lambda i: (0, 0)),
            pl.BlockSpec((NC * NPAD,), lambda i: (0,)),
        ],
        out_specs=pl.BlockSpec((BR, D), lambda i: (i, 0)),
        out_shape=jax.ShapeDtypeStruct((N, D), jnp.float32),
    )(x, W1, degp)


def _tc2_body(sp_ref, g_ref, degp_ref, b_ref, w_ref, g2_ref):
    dis = _deg_block(degp_ref, pl.program_id(0))
    t = dis * (sp_ref[0] + sp_ref[1] + g_ref[...]) + b_ref[...]
    t = jnp.maximum(t, 0.0)
    h2 = jnp.dot(t, w_ref[...], preferred_element_type=jnp.float32)
    g2_ref[...] = h2 * dis


def _tc2(sp, g1, degp, b1, W2):
    return pl.pallas_call(
        _tc2_body,
        grid=(N // BR,),
        in_specs=[
            pl.BlockSpec((NC, BR, D), lambda i: (0, i, 0)),
            pl.BlockSpec((BR, D), lambda i: (i, 0)),
            pl.BlockSpec((NC * NPAD,), lambda i: (0,)),
            pl.BlockSpec((1, D), lambda i: (0, 0)),
            pl.BlockSpec((D, D), lambda i: (0, 0)),
        ],
        out_specs=pl.BlockSpec((BR, D), lambda i: (i, 0)),
        out_shape=jax.ShapeDtypeStruct((N, D), jnp.float32),
    )(sp, g1, degp, b1, W2)


def _tc3_body(sp_ref, g_ref, degp_ref, b_ref, out_ref):
    dis = _deg_block(degp_ref, pl.program_id(0))
    out_ref[...] = dis * (sp_ref[0] + sp_ref[1] + g_ref[...]) + b_ref[...]


def _tc3(sp, g2, degp, b2):
    return pl.pallas_call(
        _tc3_body,
        grid=(N // BR,),
        in_specs=[
            pl.BlockSpec((NC, BR, D), lambda i: (0, i, 0)),
            pl.BlockSpec((BR, D), lambda i: (i, 0)),
            pl.BlockSpec((NC * NPAD,), lambda i: (0,)),
            pl.BlockSpec((1, D), lambda i: (0, 0)),
        ],
        out_specs=pl.BlockSpec((BR, D), lambda i: (i, 0)),
        out_shape=jax.ShapeDtypeStruct((N, D), jnp.float32),
    )(sp, g2, degp, b2)


def kernel(x, edge_index, W1, b1, W2, b2):
    degp = _sc_degree(edge_index)
    g1 = _tc1(x, W1, degp)
    s1 = _sc_edge_scatter(g1, edge_index)
    g2 = _tc2(s1, g1, degp, b1.reshape(1, D), W2)
    s2 = _sc_edge_scatter(g2, edge_index)
    out = _tc3(s2, g2, degp, b2.reshape(1, D))
    return out
